# baseline (speedup 1.0000x reference)
import jax
import jax.numpy as jnp
from jax.experimental import pallas as pl
from jax.experimental.pallas import tpu as pltpu

B, N, D, H = 4, 2048, 128, 128


def _fused_kernel(graph_ref, emb_ref, w1_ref, b1_ref, w2_ref, b2_ref,
                  ln_a_ref, ln_b_ref, fw1_ref, fb1_ref, fw2_ref, fb2_ref,
                  out_ref):
    eps = 1e-6
    gb = graph_ref[0].astype(jnp.bfloat16)
    emb = emb_ref[0]
    x1 = (jnp.dot(emb, w1_ref[...],
                  preferred_element_type=jnp.float32) + b1_ref[...]
          ).astype(jnp.bfloat16)
    h = jnp.dot(gb, x1, preferred_element_type=jnp.float32)
    h = jnp.maximum(h, 0.0)
    x2 = (jnp.dot(h, w2_ref[...],
                  preferred_element_type=jnp.float32) + b2_ref[...]
          ).astype(jnp.bfloat16)
    temp = jnp.dot(gb, x2, preferred_element_type=jnp.float32)
    mean = jnp.mean(temp, axis=-1, keepdims=True)
    cent = temp - mean
    var = jnp.sum(cent * cent, axis=-1, keepdims=True) / (D - 1)
    std = jnp.sqrt(var)
    normed = ln_a_ref[...] * cent / (std + eps) + ln_b_ref[...]
    num_fea = normed + emb
    ff = jnp.dot(num_fea, fw1_ref[...],
                 preferred_element_type=jnp.float32) + fb1_ref[...]
    ff = jnp.maximum(ff, 0.0)
    ff = jnp.dot(ff, fw2_ref[...],
                 preferred_element_type=jnp.float32) + fb2_ref[...]
    out_ref[0] = ff + num_fea


@jax.jit
def kernel(emb, graph, gcn_W1, gcn_b1, gcn_W2, gcn_b2, ln_a, ln_b,
           ff_W1, ff_b1, ff_W2, ff_b2):
    out = pl.pallas_call(
        _fused_kernel,
        grid=(B,),
        in_specs=[
            pl.BlockSpec((1, N, N), lambda b: (b, 0, 0)),
            pl.BlockSpec((1, N, D), lambda b: (b, 0, 0)),
            pl.BlockSpec((D, H), lambda b: (0, 0)),
            pl.BlockSpec((H,), lambda b: (0,)),
            pl.BlockSpec((H, D), lambda b: (0, 0)),
            pl.BlockSpec((D,), lambda b: (0,)),
            pl.BlockSpec((D,), lambda b: (0,)),
            pl.BlockSpec((D,), lambda b: (0,)),
            pl.BlockSpec((D, H), lambda b: (0, 0)),
            pl.BlockSpec((H,), lambda b: (0,)),
            pl.BlockSpec((H, D), lambda b: (0, 0)),
            pl.BlockSpec((D,), lambda b: (0,)),
        ],
        out_specs=pl.BlockSpec((1, N, D), lambda b: (b, 0, 0)),
        out_shape=jax.ShapeDtypeStruct((B, N, D), jnp.float32),
        compiler_params=pltpu.CompilerParams(
            vmem_limit_bytes=120 * 1024 * 1024),
    )(graph, emb, gcn_W1, gcn_b1, gcn_W2, gcn_b2, ln_a, ln_b,
      ff_W1, ff_b1, ff_W2, ff_b2)
    return out
